# Initial kernel scaffold; baseline (speedup 1.0000x reference)
import jax
import jax.numpy as jnp
from jax.experimental import pallas as pl


def kernel(vertex, vertex_features, query_pts, W1, b1, W2, b2, W3, b3, W4, b4):
    raise NotImplementedError("write your pallas kernel here")


# speedup vs baseline: 2.4518x; 2.4518x over previous
import jax
import jax.numpy as jnp
from jax import lax
from jax.experimental import pallas as pl
from jax.experimental.pallas import tpu as pltpu

N_VERT = 6890
N_PAD = 6912
Q = 16384
B = 128
K = 4
LAT = 64
HID = 128


def _body(qt_ref, qb_ref, vt_ref, tab_ref, w1f_ref, w1v_ref, b1_ref,
          w2_ref, b2_ref, w3_ref, b3_ref, w4_ref, b4_ref, out_ref):
    q = qt_ref[...]
    v = vt_ref[...]
    qb = qb_ref[...]

    d2 = (q[0][:, None] - v[0][None, :]) ** 2
    d2 = d2 + (q[1][:, None] - v[1][None, :]) ** 2
    d2 = d2 + (q[2][:, None] - v[2][None, :]) ** 2

    iota = lax.broadcasted_iota(jnp.int32, (B, N_PAD), 1)
    d = d2
    acc = jnp.zeros((B, LAT), jnp.float32)
    invsum = jnp.zeros((B, 1), jnp.float32)
    tab = tab_ref[...]
    for _ in range(K):
        m = jnp.min(d, axis=1, keepdims=True)
        ismin = d == m
        idx = jnp.min(jnp.where(ismin, iota, N_PAD), axis=1,
                      keepdims=True)
        sel = iota == idx
        d = jnp.where(sel, jnp.float32(3.0e38), d)
        oh = sel.astype(jnp.float32)
        g = jnp.dot(oh, tab, preferred_element_type=jnp.float32)
        fk = g[:, 0:LAT]
        vk = g[:, LAT:LAT + 3]
        xv = qb - vk
        pre = jnp.dot(fk, w1f_ref[...], preferred_element_type=jnp.float32)
        pre = (pre
               + xv[:, 0:1] * w1v_ref[0:1, :]
               + xv[:, 1:2] * w1v_ref[1:2, :]
               + xv[:, 2:3] * w1v_ref[2:3, :]
               + b1_ref[...])
        h = jnp.maximum(pre, 0.0)
        h = jnp.maximum(jnp.dot(h, w2_ref[...], preferred_element_type=jnp.float32)
                        + b2_ref[...], 0.0)
        h = jnp.maximum(jnp.dot(h, w3_ref[...], preferred_element_type=jnp.float32)
                        + b3_ref[...], 0.0)
        f = jnp.dot(h, w4_ref[...], preferred_element_type=jnp.float32) + b4_ref[...]
        dist = jnp.sqrt(jnp.maximum(m, 1e-12))
        inv = 1.0 / (dist + 1e-9)
        invsum = invsum + inv
        acc = acc + inv * f
    out_ref[...] = acc / invsum


def kernel(vertex, vertex_features, query_pts, W1, b1, W2, b2, W3, b3, W4, b4):
    pad = N_PAD - N_VERT
    vert_p = jnp.pad(vertex, ((0, pad), (0, 0)), constant_values=1.0e18)
    vt = vert_p.T
    qt = query_pts.T
    feat_p = jnp.pad(vertex_features, ((0, pad), (0, 0)))
    vert_tab = jnp.pad(vertex, ((0, pad), (0, 0)))
    tab = jnp.concatenate(
        [feat_p, vert_tab, jnp.zeros((N_PAD, 128 - LAT - 3), jnp.float32)],
        axis=1)
    w1f = W1[:LAT, :]
    w1v = W1[LAT:, :]

    grid = Q // B
    full = lambda shape: pl.BlockSpec(shape, lambda i: (0,) * len(shape))
    out = pl.pallas_call(
        _body,
        grid=(grid,),
        in_specs=[
            pl.BlockSpec((3, B), lambda i: (0, i)),
            pl.BlockSpec((B, 3), lambda i: (i, 0)),
            full((3, N_PAD)),
            full((N_PAD, 128)),
            full((LAT, HID)),
            full((3, HID)),
            full((1, HID)),
            full((HID, HID)),
            full((1, HID)),
            full((HID, HID)),
            full((1, HID)),
            full((HID, LAT)),
            full((1, LAT)),
        ],
        out_specs=pl.BlockSpec((B, LAT), lambda i: (i, 0)),
        out_shape=jax.ShapeDtypeStruct((Q, LAT), jnp.float32),
    )(qt, query_pts, vt, tab, w1f, w1v, b1.reshape(1, HID),
      W2, b2.reshape(1, HID), W3, b3.reshape(1, HID), W4, b4.reshape(1, LAT))
    return out
